# Initial kernel scaffold; baseline (speedup 1.0000x reference)
import functools

import jax
import jax.numpy as jnp
from jax import lax
from jax.experimental import pallas as pl
from jax.experimental.pallas import tpu as pltpu
from jax.experimental.pallas import tpu_sc as plsc

N = 10000
E = 160000
H = 256
HH = 128
OUT = 64
ROWS = 1000
C = 80
NT = 16
EPT = E // NT
NCH = N // C

_PREC = jax.lax.Precision.HIGHEST


def _tc_first_body(x_ref, w_ref, asv_ref, adv_ref, h_ref, as_ref, ad_ref):
    h = jnp.dot(x_ref[...], w_ref[...], precision=_PREC)
    h_ref[0] = h[:, :HH]
    h_ref[1] = h[:, HH:]
    as_ref[...] = jnp.sum(h * asv_ref[...], axis=1, keepdims=True)
    ad_ref[...] = jnp.sum(h * adv_ref[...], axis=1, keepdims=True)


def _tc_first(x, W, asv, adv):
    grid = (N // ROWS,)
    return pl.pallas_call(
        _tc_first_body,
        grid=grid,
        in_specs=[
            pl.BlockSpec((ROWS, H), lambda i: (i, 0)),
            pl.BlockSpec((H, H), lambda i: (0, 0)),
            pl.BlockSpec((1, H), lambda i: (0, 0)),
            pl.BlockSpec((1, H), lambda i: (0, 0)),
        ],
        out_specs=[
            pl.BlockSpec((2, ROWS, HH), lambda i: (0, i, 0)),
            pl.BlockSpec((ROWS, 1), lambda i: (i, 0)),
            pl.BlockSpec((ROWS, 1), lambda i: (i, 0)),
        ],
        out_shape=[
            jax.ShapeDtypeStruct((2, N, HH), jnp.float32),
            jax.ShapeDtypeStruct((N, 1), jnp.float32),
            jax.ShapeDtypeStruct((N, 1), jnp.float32),
        ],
    )(x, W, asv, adv)


def _node_update(agg_ref, s_ref, b_ref):
    d = s_ref[...] + 1e-16
    x0 = jnp.maximum(agg_ref[0] / d + b_ref[:, :HH], 0.0)
    x1 = jnp.maximum(agg_ref[1] / d + b_ref[:, HH:], 0.0)
    return jnp.concatenate([x0, x1], axis=1)


def _tc_mid_body(agg_ref, s_ref, b_ref, w_ref, asv_ref, adv_ref,
                 x_ref, h_ref, as_ref, ad_ref):
    x = _node_update(agg_ref, s_ref, b_ref)
    x_ref[...] = x
    h = jnp.dot(x, w_ref[...], precision=_PREC)
    h_ref[0] = h[:, :HH]
    h_ref[1] = h[:, HH:]
    as_ref[...] = jnp.sum(h * asv_ref[...], axis=1, keepdims=True)
    ad_ref[...] = jnp.sum(h * adv_ref[...], axis=1, keepdims=True)


def _tc_mid(agg, s, b, W, asv, adv):
    grid = (N // ROWS,)
    return pl.pallas_call(
        _tc_mid_body,
        grid=grid,
        in_specs=[
            pl.BlockSpec((2, ROWS, HH), lambda i: (0, i, 0)),
            pl.BlockSpec((ROWS, 1), lambda i: (i, 0)),
            pl.BlockSpec((1, H), lambda i: (0, 0)),
            pl.BlockSpec((H, H), lambda i: (0, 0)),
            pl.BlockSpec((1, H), lambda i: (0, 0)),
            pl.BlockSpec((1, H), lambda i: (0, 0)),
        ],
        out_specs=[
            pl.BlockSpec((ROWS, H), lambda i: (i, 0)),
            pl.BlockSpec((2, ROWS, HH), lambda i: (0, i, 0)),
            pl.BlockSpec((ROWS, 1), lambda i: (i, 0)),
            pl.BlockSpec((ROWS, 1), lambda i: (i, 0)),
        ],
        out_shape=[
            jax.ShapeDtypeStruct((N, H), jnp.float32),
            jax.ShapeDtypeStruct((2, N, HH), jnp.float32),
            jax.ShapeDtypeStruct((N, 1), jnp.float32),
            jax.ShapeDtypeStruct((N, 1), jnp.float32),
        ],
    )(agg, s, b, W, asv, adv)


def _tc_final_body(x1_ref, x2_ref, agg_ref, s_ref, b_ref,
                   w1_ref, w2_ref, w3_ref, bo_ref, o_ref):
    x3 = _node_update(agg_ref, s_ref, b_ref)
    o = jnp.dot(x1_ref[...], w1_ref[...], precision=_PREC)
    o += jnp.dot(x2_ref[...], w2_ref[...], precision=_PREC)
    o += jnp.dot(x3, w3_ref[...], precision=_PREC)
    o_ref[...] = o + bo_ref[...]


def _tc_final(x1, x2, agg, s, b, W1, W2, W3, bo):
    grid = (N // ROWS,)
    return pl.pallas_call(
        _tc_final_body,
        grid=grid,
        in_specs=[
            pl.BlockSpec((ROWS, H), lambda i: (i, 0)),
            pl.BlockSpec((ROWS, H), lambda i: (i, 0)),
            pl.BlockSpec((2, ROWS, HH), lambda i: (0, i, 0)),
            pl.BlockSpec((ROWS, 1), lambda i: (i, 0)),
            pl.BlockSpec((1, H), lambda i: (0, 0)),
            pl.BlockSpec((H, OUT), lambda i: (0, 0)),
            pl.BlockSpec((H, OUT), lambda i: (0, 0)),
            pl.BlockSpec((H, OUT), lambda i: (0, 0)),
            pl.BlockSpec((1, OUT), lambda i: (0, 0)),
        ],
        out_specs=pl.BlockSpec((ROWS, OUT), lambda i: (i, 0)),
        out_shape=jax.ShapeDtypeStruct((N, OUT), jnp.float32),
    )(x1, x2, agg, s, b, W1, W2, W3, bo)


def _sc_layer_body(h_hbm, src_hbm, dst_hbm, asrc_hbm, adst_hbm,
                   agg_hbm, s_hbm,
                   asrc_v, adst_v, srcb, dstb, exb, rows,
                   acc_sh, s_sh):
    c = lax.axis_index("c")
    t = lax.axis_index("s")

    zv = jnp.zeros((16,), jnp.float32)

    @pl.loop(0, C)
    def _(i):
        for j in range(HH // 16):
            rows[i, pl.ds(j * 16, 16)] = zv

    @pl.loop(0, C // 16)
    def _(i):
        exb[pl.ds(i * 16, 16)] = zv

    @pl.loop(0, NCH)
    def _(k):
        @pl.when(lax.rem(k, NT) == t)
        def _():
            pltpu.sync_copy(rows, acc_sh.at[pl.ds(k * C, C)])
            pltpu.sync_copy(exb, s_sh.at[pl.ds(k * C, C)])

    pltpu.sync_copy(asrc_hbm, asrc_v)
    pltpu.sync_copy(adst_hbm, adst_v)
    plsc.subcore_barrier()

    base = t * EPT

    @pl.loop(0, EPT // C)
    def _(k):
        off = base + k * C
        pltpu.sync_copy(src_hbm.at[pl.ds(off, C)], srcb)
        pltpu.sync_copy(dst_hbm.at[pl.ds(off, C)], dstb)
        for j in range(C // 16):
            sl = pl.ds(j * 16, 16)
            si = srcb[sl]
            di = dstb[sl]
            e = plsc.load_gather(asrc_v, [si]) + plsc.load_gather(adst_v, [di])
            e = jnp.maximum(e, 0.2 * e)
            exb[sl] = jnp.exp(e)
        pltpu.sync_copy(h_hbm.at[c].at[srcb], rows)

        @pl.loop(0, C)
        def _(i):
            w = exb[i]
            for j in range(HH // 16):
                sl = pl.ds(j * 16, 16)
                rows[i, sl] = rows[i, sl] * w

        pltpu.sync_copy(rows, acc_sh.at[dstb], add=True)
        pltpu.sync_copy(exb, s_sh.at[dstb], add=True)

    plsc.subcore_barrier()

    @pl.loop(0, NCH)
    def _(k):
        @pl.when(lax.rem(k, NT) == t)
        def _():
            pltpu.sync_copy(acc_sh.at[pl.ds(k * C, C)],
                            agg_hbm.at[c].at[pl.ds(k * C, C)])
            pltpu.sync_copy(s_sh.at[pl.ds(k * C, C)],
                            s_hbm.at[c].at[pl.ds(k * C, C)])


_SC_MESH = plsc.VectorSubcoreMesh(core_axis_name="c", subcore_axis_name="s")


@functools.partial(
    pl.kernel,
    out_type=[
        jax.ShapeDtypeStruct((2, N, HH), jnp.float32),
        jax.ShapeDtypeStruct((2, N), jnp.float32),
    ],
    mesh=_SC_MESH,
    scratch_types=[
        pltpu.VMEM((N,), jnp.float32),
        pltpu.VMEM((N,), jnp.float32),
        pltpu.VMEM((C,), jnp.int32),
        pltpu.VMEM((C,), jnp.int32),
        pltpu.VMEM((C,), jnp.float32),
        pltpu.VMEM((C, HH), jnp.float32),
        pltpu.VMEM_SHARED((N, HH), jnp.float32),
        pltpu.VMEM_SHARED((N,), jnp.float32),
    ],
)
def _sc_layer(h_hbm, src_hbm, dst_hbm, asrc_hbm, adst_hbm, agg_hbm, s_hbm,
              asrc_v, adst_v, srcb, dstb, exb, rows, acc_sh, s_sh):
    _sc_layer_body(h_hbm, src_hbm, dst_hbm, asrc_hbm, adst_hbm,
                   agg_hbm, s_hbm,
                   asrc_v, adst_v, srcb, dstb, exb, rows, acc_sh, s_sh)


def kernel(x, edge_index, W0, as0, ad0, b0, W1, as1, ad1, b1,
           W2, as2, ad2, b2, Wout, bout):
    src = edge_index[0]
    dst = edge_index[1]

    h0, a0s, a0d = _tc_first(x, W0, as0.reshape(1, H), ad0.reshape(1, H))
    agg0, s0 = _sc_layer(h0, src, dst, a0s.reshape(N), a0d.reshape(N))

    x1, h1, a1s, a1d = _tc_mid(agg0, s0[0].reshape(N, 1), b0.reshape(1, H),
                               W1, as1.reshape(1, H), ad1.reshape(1, H))
    agg1, s1 = _sc_layer(h1, src, dst, a1s.reshape(N), a1d.reshape(N))

    x2, h2, a2s, a2d = _tc_mid(agg1, s1[0].reshape(N, 1), b1.reshape(1, H),
                               W2, as2.reshape(1, H), ad2.reshape(1, H))
    agg2, s2 = _sc_layer(h2, src, dst, a2s.reshape(N), a2d.reshape(N))

    return _tc_final(x1, x2, agg2, s2[0].reshape(N, 1), b2.reshape(1, H),
                     Wout[:H], Wout[H:2 * H], Wout[2 * H:], bout.reshape(1, OUT))


# speedup vs baseline: 11.4684x; 11.4684x over previous
import dataclasses
import functools

import jax
import jax.numpy as jnp
from jax import lax
from jax.experimental import pallas as pl
from jax.experimental.pallas import tpu as pltpu
from jax.experimental.pallas import tpu_sc as plsc

N = 10000
NP = 10240
E = 160000
H = 256
HH = 128
OUT = 64
ROWS = 1000
C = 128
NT = 16
KE = E // C
KN = NP // C

_PREC = jax.lax.Precision.HIGHEST


def _tc_first_body(x_ref, w_ref, asv_ref, adv_ref, h_ref, as_ref, ad_ref):
    h = jnp.dot(x_ref[...], w_ref[...], precision=_PREC)
    h_ref[0] = h[:, :HH]
    h_ref[1] = h[:, HH:]
    as_ref[...] = jnp.sum(h * asv_ref[...], axis=1, keepdims=True)
    ad_ref[...] = jnp.sum(h * adv_ref[...], axis=1, keepdims=True)


def _tc_first(x, W, asv, adv):
    grid = (N // ROWS,)
    return pl.pallas_call(
        _tc_first_body,
        grid=grid,
        in_specs=[
            pl.BlockSpec((ROWS, H), lambda i: (i, 0)),
            pl.BlockSpec((H, H), lambda i: (0, 0)),
            pl.BlockSpec((1, H), lambda i: (0, 0)),
            pl.BlockSpec((1, H), lambda i: (0, 0)),
        ],
        out_specs=[
            pl.BlockSpec((2, ROWS, HH), lambda i: (0, i, 0)),
            pl.BlockSpec((ROWS, 1), lambda i: (i, 0)),
            pl.BlockSpec((ROWS, 1), lambda i: (i, 0)),
        ],
        out_shape=[
            jax.ShapeDtypeStruct((2, N, HH), jnp.float32),
            jax.ShapeDtypeStruct((N, 1), jnp.float32),
            jax.ShapeDtypeStruct((N, 1), jnp.float32),
        ],
    )(x, W, asv, adv)


def _node_update(agg_ref, s_ref, b_ref):
    d = s_ref[...] + 1e-16
    x0 = jnp.maximum(agg_ref[0] / d + b_ref[:, :HH], 0.0)
    x1 = jnp.maximum(agg_ref[1] / d + b_ref[:, HH:], 0.0)
    return jnp.concatenate([x0, x1], axis=1)


def _tc_mid_body(agg_ref, s_ref, b_ref, w_ref, asv_ref, adv_ref,
                 x_ref, h_ref, as_ref, ad_ref):
    x = _node_update(agg_ref, s_ref, b_ref)
    x_ref[...] = x
    h = jnp.dot(x, w_ref[...], precision=_PREC)
    h_ref[0] = h[:, :HH]
    h_ref[1] = h[:, HH:]
    as_ref[...] = jnp.sum(h * asv_ref[...], axis=1, keepdims=True)
    ad_ref[...] = jnp.sum(h * adv_ref[...], axis=1, keepdims=True)


def _tc_mid(agg, s, b, W, asv, adv):
    grid = (N // ROWS,)
    return pl.pallas_call(
        _tc_mid_body,
        grid=grid,
        in_specs=[
            pl.BlockSpec((2, ROWS, HH), lambda i: (0, i, 0)),
            pl.BlockSpec((ROWS, 1), lambda i: (i, 0)),
            pl.BlockSpec((1, H), lambda i: (0, 0)),
            pl.BlockSpec((H, H), lambda i: (0, 0)),
            pl.BlockSpec((1, H), lambda i: (0, 0)),
            pl.BlockSpec((1, H), lambda i: (0, 0)),
        ],
        out_specs=[
            pl.BlockSpec((ROWS, H), lambda i: (i, 0)),
            pl.BlockSpec((2, ROWS, HH), lambda i: (0, i, 0)),
            pl.BlockSpec((ROWS, 1), lambda i: (i, 0)),
            pl.BlockSpec((ROWS, 1), lambda i: (i, 0)),
        ],
        out_shape=[
            jax.ShapeDtypeStruct((N, H), jnp.float32),
            jax.ShapeDtypeStruct((2, N, HH), jnp.float32),
            jax.ShapeDtypeStruct((N, 1), jnp.float32),
            jax.ShapeDtypeStruct((N, 1), jnp.float32),
        ],
    )(agg, s, b, W, asv, adv)


def _tc_final_body(x1_ref, x2_ref, agg_ref, s_ref, b_ref,
                   w1_ref, w2_ref, w3_ref, bo_ref, o_ref):
    x3 = _node_update(agg_ref, s_ref, b_ref)
    o = jnp.dot(x1_ref[...], w1_ref[...], precision=_PREC)
    o += jnp.dot(x2_ref[...], w2_ref[...], precision=_PREC)
    o += jnp.dot(x3, w3_ref[...], precision=_PREC)
    o_ref[...] = o + bo_ref[...]


def _tc_final(x1, x2, agg, s, b, W1, W2, W3, bo):
    grid = (N // ROWS,)
    return pl.pallas_call(
        _tc_final_body,
        grid=grid,
        in_specs=[
            pl.BlockSpec((ROWS, H), lambda i: (i, 0)),
            pl.BlockSpec((ROWS, H), lambda i: (i, 0)),
            pl.BlockSpec((2, ROWS, HH), lambda i: (0, i, 0)),
            pl.BlockSpec((ROWS, 1), lambda i: (i, 0)),
            pl.BlockSpec((1, H), lambda i: (0, 0)),
            pl.BlockSpec((H, OUT), lambda i: (0, 0)),
            pl.BlockSpec((H, OUT), lambda i: (0, 0)),
            pl.BlockSpec((H, OUT), lambda i: (0, 0)),
            pl.BlockSpec((1, OUT), lambda i: (0, 0)),
        ],
        out_specs=pl.BlockSpec((ROWS, OUT), lambda i: (i, 0)),
        out_shape=jax.ShapeDtypeStruct((N, OUT), jnp.float32),
    )(x1, x2, agg, s, b, W1, W2, W3, bo)


def _sc_layer_body(h_hbm, src_hbm, dst_hbm, asrc_hbm, adst_hbm,
                   agg_hbm, s_hbm,
                   asrc_v, adst_v, srcb, dstb, exb, rows,
                   acc_sh, s_sh):
    c = lax.axis_index("c")
    t = lax.axis_index("s")

    zv = jnp.zeros((16,), jnp.float32)

    @pl.loop(0, C)
    def _(i):
        for j in range(HH // 16):
            rows[i, pl.ds(j * 16, 16)] = zv

    @pl.loop(0, C // 16)
    def _(i):
        exb[pl.ds(i * 16, 16)] = zv

    @pl.loop(0, KN // NT)
    def _(i):
        off = (t * (KN // NT) + i) * C
        pltpu.sync_copy(rows, acc_sh.at[pl.ds(off, C)])
        pltpu.sync_copy(exb, s_sh.at[pl.ds(off, C)])

    pltpu.sync_copy(asrc_hbm, asrc_v)
    pltpu.sync_copy(adst_hbm, adst_v)
    plsc.subcore_barrier()

    @pl.loop(0, pl.cdiv(KE, NT))
    def _(j):
        k = j * NT + t

        @pl.when(k < KE)
        def _():
            off = k * C
            pltpu.sync_copy(src_hbm.at[pl.ds(off, C)], srcb)
            pltpu.sync_copy(dst_hbm.at[pl.ds(off, C)], dstb)
            for jj in range(C // 16):
                sl = pl.ds(jj * 16, 16)
                si = srcb[sl]
                di = dstb[sl]
                e = (plsc.load_gather(asrc_v, [si])
                     + plsc.load_gather(adst_v, [di]))
                e = jnp.maximum(e, 0.2 * e)
                exb[sl] = jnp.exp(e)
            pltpu.sync_copy(h_hbm.at[c].at[srcb], rows)

            @pl.loop(0, C)
            def _(i):
                wv = plsc.load_gather(exb, [jnp.full((16,), i, jnp.int32)])
                for jj in range(HH // 16):
                    sl = pl.ds(jj * 16, 16)
                    rows[i, sl] = rows[i, sl] * wv

            pltpu.sync_copy(rows, acc_sh.at[dstb], add=True)
            pltpu.sync_copy(exb, s_sh.at[dstb], add=True)

    plsc.subcore_barrier()

    @pl.loop(0, KN // NT)
    def _(i):
        off = (t * (KN // NT) + i) * C
        pltpu.sync_copy(acc_sh.at[pl.ds(off, C)],
                        agg_hbm.at[c].at[pl.ds(off, C)])
        pltpu.sync_copy(s_sh.at[pl.ds(off, C)],
                        s_hbm.at[c].at[pl.ds(off, C)])


_SC_MESH = plsc.VectorSubcoreMesh(core_axis_name="c", subcore_axis_name="s")

_SC_PARAMS = pltpu.CompilerParams()
if "needs_layout_passes" in pltpu.CompilerParams.__dataclass_fields__:
    _SC_PARAMS = dataclasses.replace(_SC_PARAMS, needs_layout_passes=False)


@functools.partial(
    pl.kernel,
    compiler_params=_SC_PARAMS,
    out_type=[
        jax.ShapeDtypeStruct((2, NP, HH), jnp.float32),
        jax.ShapeDtypeStruct((2, NP), jnp.float32),
    ],
    mesh=_SC_MESH,
    scratch_types=[
        pltpu.VMEM((N,), jnp.float32),
        pltpu.VMEM((N,), jnp.float32),
        pltpu.VMEM((C,), jnp.int32),
        pltpu.VMEM((C,), jnp.int32),
        pltpu.VMEM((C,), jnp.float32),
        pltpu.VMEM((C, HH), jnp.float32),
        pltpu.VMEM_SHARED((NP, HH), jnp.float32),
        pltpu.VMEM_SHARED((NP,), jnp.float32),
    ],
)
def _sc_layer(h_hbm, src_hbm, dst_hbm, asrc_hbm, adst_hbm, agg_hbm, s_hbm,
              asrc_v, adst_v, srcb, dstb, exb, rows, acc_sh, s_sh):
    _sc_layer_body(h_hbm, src_hbm, dst_hbm, asrc_hbm, adst_hbm,
                   agg_hbm, s_hbm,
                   asrc_v, adst_v, srcb, dstb, exb, rows, acc_sh, s_sh)


def kernel(x, edge_index, W0, as0, ad0, b0, W1, as1, ad1, b1,
           W2, as2, ad2, b2, Wout, bout):
    src = edge_index[0]
    dst = edge_index[1]

    h0, a0s, a0d = _tc_first(x, W0, as0.reshape(1, H), ad0.reshape(1, H))
    agg0, s0 = _sc_layer(h0, src, dst, a0s.reshape(N), a0d.reshape(N))

    x1, h1, a1s, a1d = _tc_mid(agg0, s0[0].reshape(NP, 1), b0.reshape(1, H),
                               W1, as1.reshape(1, H), ad1.reshape(1, H))
    agg1, s1 = _sc_layer(h1, src, dst, a1s.reshape(N), a1d.reshape(N))

    x2, h2, a2s, a2d = _tc_mid(agg1, s1[0].reshape(NP, 1), b1.reshape(1, H),
                               W2, as2.reshape(1, H), ad2.reshape(1, H))
    agg2, s2 = _sc_layer(h2, src, dst, a2s.reshape(N), a2d.reshape(N))

    return _tc_final(x1, x2, agg2, s2[0].reshape(NP, 1), b2.reshape(1, H),
                     Wout[:H], Wout[H:2 * H], Wout[2 * H:], bout.reshape(1, OUT))
